# Initial kernel scaffold; baseline (speedup 1.0000x reference)
import jax
import jax.numpy as jnp
from jax.experimental import pallas as pl


def kernel(x, neighbors, max_iter, last_layer):
    raise NotImplementedError("write your pallas kernel here")


# speedup vs baseline: 6.4616x; 6.4616x over previous
import functools

import jax
import jax.numpy as jnp
from jax import lax
from jax.experimental import pallas as pl
from jax.experimental.pallas import tpu as pltpu
from jax.experimental.pallas import tpu_sc as plsc

_K = 8
_DD = 16
_NC, _NS = 2, 16
_NW = _NC * _NS
_ITERS = 6


def _seg_matrix(d, dtype):
    lane = lax.broadcasted_iota(jnp.int32, (d, _K), 0)
    cap = lax.broadcasted_iota(jnp.int32, (d, _K), 1)
    return (lane // _DD == cap).astype(dtype)


def _cap_normalize(u, s_mat):
    sq = jnp.dot(u * u, s_mat, preferred_element_type=jnp.float32)
    scale = 1.0 / jnp.maximum(jnp.sqrt(sq), 1e-12)
    return u * jnp.dot(scale, s_mat.T, preferred_element_type=jnp.float32)


def _normalize_body(x_ref, o_ref):
    x = x_ref[...]
    o_ref[...] = _cap_normalize(x, _seg_matrix(x.shape[-1], x.dtype))


def _routing_body(z_ref, x_ref, o_ref):
    z = z_ref[...]
    x = x_ref[...]
    b, m, d = z.shape
    s_mat = _seg_matrix(d, x.dtype)
    zf = z.reshape(b * m, d)

    u = jnp.sum(z, axis=1) * (1.0 / _K) + x
    u = _cap_normalize(u, s_mat)
    for it in range(1, _ITERS):
        t = (z * u[:, None, :]).reshape(b * m, d)
        p = jnp.dot(t, s_mat, preferred_element_type=jnp.float32)
        p = p - jnp.max(p, axis=1, keepdims=True)
        e = jnp.exp(p)
        p = e / jnp.sum(e, axis=1, keepdims=True)
        pe = jnp.dot(p, s_mat.T, preferred_element_type=jnp.float32)
        u = jnp.sum((zf * pe).reshape(b, m, d), axis=1) + x
        if it < _ITERS - 1:
            u = _cap_normalize(u, s_mat)
    o_ref[...] = u


def _normalize_call(x):
    n, d = x.shape
    bn = 2000
    return pl.pallas_call(
        _normalize_body,
        grid=(n // bn,),
        in_specs=[pl.BlockSpec((bn, d), lambda i: (i, 0))],
        out_specs=pl.BlockSpec((bn, d), lambda i: (i, 0)),
        out_shape=jax.ShapeDtypeStruct((n, d), x.dtype),
    )(x)


def _routing_call(z3, x_norm):
    n, m, d = z3.shape
    bn = 200
    return pl.pallas_call(
        _routing_body,
        grid=(n // bn,),
        in_specs=[
            pl.BlockSpec((bn, m, d), lambda i: (i, 0, 0)),
            pl.BlockSpec((bn, d), lambda i: (i, 0)),
        ],
        out_specs=pl.BlockSpec((bn, d), lambda i: (i, 0)),
        out_shape=jax.ShapeDtypeStruct((n, d), x_norm.dtype),
    )(z3, x_norm)


def _gather_call(table, idx):
    e_total = idx.shape[0]
    per_w = e_total // _NW
    chunk = 400
    steps = per_w // chunk
    d = table.shape[1]
    mesh = plsc.VectorSubcoreMesh(core_axis_name="c", subcore_axis_name="s")

    @functools.partial(
        pl.kernel,
        out_type=jax.ShapeDtypeStruct((e_total, d), table.dtype),
        mesh=mesh,
        scratch_types=[
            pltpu.VMEM((chunk,), jnp.int32),
            pltpu.VMEM((chunk, d), table.dtype),
            pltpu.SemaphoreType.DMA,
        ],
    )
    def gather_kernel(table_hbm, idx_hbm, out_hbm, idx_v, rows_v, sem):
        wid = lax.axis_index("s") * _NC + lax.axis_index("c")
        base = wid * per_w

        def body(i, carry):
            off = base + i * chunk
            pltpu.sync_copy(idx_hbm.at[pl.ds(off, chunk)], idx_v)
            pltpu.async_copy(table_hbm.at[idx_v], rows_v, sem).wait()
            pltpu.sync_copy(rows_v, out_hbm.at[pl.ds(off, chunk)])
            return carry

        lax.fori_loop(0, steps, body, 0)

    return gather_kernel(table, idx)


def kernel(x, neighbors, max_iter, last_layer):
    del max_iter, last_layer
    n, d = x.shape
    m = neighbors.shape[0] // n
    x_norm = _normalize_call(x)
    table = jnp.concatenate(
        [x_norm, jnp.zeros((8, d), dtype=x_norm.dtype)], axis=0)
    z = _gather_call(table, neighbors)
    u = _routing_call(z.reshape(n, m, d), x_norm)
    return u


# speedup vs baseline: 7.0629x; 1.0931x over previous
import functools

import jax
import jax.numpy as jnp
from jax import lax
from jax.experimental import pallas as pl
from jax.experimental.pallas import tpu as pltpu
from jax.experimental.pallas import tpu_sc as plsc

_K = 8
_DD = 16
_NC, _NS = 2, 16
_NW = _NC * _NS
_ITERS = 6


def _seg_matrix(d, dtype):
    lane = lax.broadcasted_iota(jnp.int32, (d, _K), 0)
    cap = lax.broadcasted_iota(jnp.int32, (d, _K), 1)
    return (lane // _DD == cap).astype(dtype)


def _cap_normalize(u, s_mat):
    sq = jnp.dot(u * u, s_mat, preferred_element_type=jnp.float32)
    scale = lax.rsqrt(jnp.maximum(sq, 1e-24))
    return u * jnp.dot(scale, s_mat.T, preferred_element_type=jnp.float32)


def _normalize_body(x_ref, o_ref):
    x = x_ref[...]
    o_ref[...] = _cap_normalize(x, _seg_matrix(x.shape[-1], x.dtype))


def _routing_body(z_ref, x_ref, o_ref):
    z = z_ref[...]
    x = x_ref[...]
    b, m, d = z.shape
    s_mat = _seg_matrix(d, x.dtype)
    s_log2e = s_mat * 1.4426950408889634
    ones_k = jnp.ones((_K, _K), x.dtype)
    zf = z.reshape(b * m, d)

    u = jnp.sum(z, axis=1) * (1.0 / _K) + x
    u = _cap_normalize(u, s_mat)
    for it in range(1, _ITERS):
        t = (z * u[:, None, :]).reshape(b * m, d)
        p = jnp.dot(t, s_log2e, preferred_element_type=jnp.float32)
        e = jnp.exp2(p)
        den = jnp.dot(e, ones_k, preferred_element_type=jnp.float32)
        pn = e / den
        pe = jnp.dot(pn, s_mat.T, preferred_element_type=jnp.float32)
        u = jnp.sum((zf * pe).reshape(b, m, d), axis=1) + x
        if it < _ITERS - 1:
            u = _cap_normalize(u, s_mat)
    o_ref[...] = u


def _normalize_call(x):
    n, d = x.shape
    bn = 2000
    return pl.pallas_call(
        _normalize_body,
        grid=(n // bn,),
        in_specs=[pl.BlockSpec((bn, d), lambda i: (i, 0))],
        out_specs=pl.BlockSpec((bn, d), lambda i: (i, 0)),
        out_shape=jax.ShapeDtypeStruct((n, d), x.dtype),
    )(x)


def _routing_call(z3, x_norm):
    n, m, d = z3.shape
    bn = 200
    return pl.pallas_call(
        _routing_body,
        grid=(n // bn,),
        in_specs=[
            pl.BlockSpec((bn, m, d), lambda i: (i, 0, 0)),
            pl.BlockSpec((bn, d), lambda i: (i, 0)),
        ],
        out_specs=pl.BlockSpec((bn, d), lambda i: (i, 0)),
        out_shape=jax.ShapeDtypeStruct((n, d), x_norm.dtype),
    )(z3, x_norm)


def _gather_call(table, idx):
    e_total = idx.shape[0]
    per_w = e_total // _NW
    chunk = 400
    steps = per_w // chunk
    d = table.shape[1]
    mesh = plsc.VectorSubcoreMesh(core_axis_name="c", subcore_axis_name="s")

    @functools.partial(
        pl.kernel,
        out_type=jax.ShapeDtypeStruct((e_total, d), table.dtype),
        mesh=mesh,
        scratch_types=[
            pltpu.VMEM((chunk,), jnp.int32),
            pltpu.VMEM((chunk, d), table.dtype),
            pltpu.SemaphoreType.DMA,
        ],
    )
    def gather_kernel(table_hbm, idx_hbm, out_hbm, idx_v, rows_v, sem):
        wid = lax.axis_index("s") * _NC + lax.axis_index("c")
        base = wid * per_w

        def body(i, carry):
            off = base + i * chunk
            pltpu.sync_copy(idx_hbm.at[pl.ds(off, chunk)], idx_v)
            pltpu.async_copy(table_hbm.at[idx_v], rows_v, sem).wait()
            pltpu.sync_copy(rows_v, out_hbm.at[pl.ds(off, chunk)])
            return carry

        lax.fori_loop(0, steps, body, 0)

    return gather_kernel(table, idx)


def kernel(x, neighbors, max_iter, last_layer):
    del max_iter, last_layer
    n, d = x.shape
    m = neighbors.shape[0] // n
    x_norm = _normalize_call(x)
    table = jnp.concatenate(
        [x_norm, jnp.zeros((8, d), dtype=x_norm.dtype)], axis=0)
    z = _gather_call(table, neighbors)
    u = _routing_call(z.reshape(n, m, d), x_norm)
    return u


# speedup vs baseline: 7.4125x; 1.0495x over previous
import functools

import jax
import jax.numpy as jnp
from jax import lax
from jax.experimental import pallas as pl
from jax.experimental.pallas import tpu as pltpu
from jax.experimental.pallas import tpu_sc as plsc

_K = 8
_DD = 16
_NC, _NS = 2, 16
_NW = _NC * _NS
_ITERS = 6


def _seg_matrix(d, dtype):
    lane = lax.broadcasted_iota(jnp.int32, (d, _K), 0)
    cap = lax.broadcasted_iota(jnp.int32, (d, _K), 1)
    return (lane // _DD == cap).astype(dtype)


def _cap_normalize(u, s_mat):
    sq = jnp.dot(u * u, s_mat, preferred_element_type=jnp.float32)
    scale = lax.rsqrt(jnp.maximum(sq, 1e-24))
    return u * jnp.dot(scale, s_mat.T, preferred_element_type=jnp.float32)


def _normalize_body(x_ref, o_ref):
    x = x_ref[...]
    o_ref[...] = _cap_normalize(x, _seg_matrix(x.shape[-1], x.dtype))


def _transpose_body(n_ref, o_ref):
    o_ref[...] = n_ref[...].T


def _routing_body(z_ref, x_ref, o_ref):
    z = z_ref[...]
    x = x_ref[...]
    m, b, d = z.shape
    s_mat = _seg_matrix(d, x.dtype)
    s_log2e = s_mat * 1.4426950408889634
    ones_k = jnp.ones((_K, _K), x.dtype)
    zf = z.reshape(m * b, d)

    u = jnp.sum(z, axis=0) * (1.0 / _K) + x
    u = _cap_normalize(u, s_mat)
    for it in range(1, _ITERS):
        t = (z * u[None, :, :]).reshape(m * b, d)
        p = jnp.dot(t, s_log2e, preferred_element_type=jnp.float32)
        e = jnp.exp2(p)
        den = jnp.dot(e, ones_k, preferred_element_type=jnp.float32)
        pn = e / den
        pe = jnp.dot(pn, s_mat.T, preferred_element_type=jnp.float32)
        u = jnp.sum((zf * pe).reshape(m, b, d), axis=0) + x
        if it < _ITERS - 1:
            u = _cap_normalize(u, s_mat)
    o_ref[...] = u


def _normalize_call(x):
    n, d = x.shape
    bn = 2000
    return pl.pallas_call(
        _normalize_body,
        grid=(n // bn,),
        in_specs=[pl.BlockSpec((bn, d), lambda i: (i, 0))],
        out_specs=pl.BlockSpec((bn, d), lambda i: (i, 0)),
        out_shape=jax.ShapeDtypeStruct((n, d), x.dtype),
    )(x)


def _transpose_call(nbr2):
    n, m = nbr2.shape
    return pl.pallas_call(
        _transpose_body,
        grid=(1,),
        in_specs=[pl.BlockSpec((n, m), lambda i: (0, 0))],
        out_specs=pl.BlockSpec((m, n), lambda i: (0, 0)),
        out_shape=jax.ShapeDtypeStruct((m, n), nbr2.dtype),
    )(nbr2)


def _routing_call(z3, x_norm):
    m, n, d = z3.shape
    bn = 400
    return pl.pallas_call(
        _routing_body,
        grid=(n // bn,),
        in_specs=[
            pl.BlockSpec((m, bn, d), lambda i: (0, i, 0)),
            pl.BlockSpec((bn, d), lambda i: (i, 0)),
        ],
        out_specs=pl.BlockSpec((bn, d), lambda i: (i, 0)),
        out_shape=jax.ShapeDtypeStruct((n, d), x_norm.dtype),
    )(z3, x_norm)


def _gather_call(table, idx):
    e_total = idx.shape[0]
    per_w = e_total // _NW
    chunk = 400
    steps = per_w // chunk
    d = table.shape[1]
    mesh = plsc.VectorSubcoreMesh(core_axis_name="c", subcore_axis_name="s")

    @functools.partial(
        pl.kernel,
        out_type=jax.ShapeDtypeStruct((e_total, d), table.dtype),
        mesh=mesh,
        scratch_types=[
            pltpu.VMEM((chunk,), jnp.int32),
            pltpu.VMEM((chunk, d), table.dtype),
            pltpu.SemaphoreType.DMA,
        ],
    )
    def gather_kernel(table_hbm, idx_hbm, out_hbm, idx_v, rows_v, sem):
        wid = lax.axis_index("s") * _NC + lax.axis_index("c")
        base = wid * per_w

        def body(i, carry):
            off = base + i * chunk
            pltpu.sync_copy(idx_hbm.at[pl.ds(off, chunk)], idx_v)
            pltpu.async_copy(table_hbm.at[idx_v], rows_v, sem).wait()
            pltpu.sync_copy(rows_v, out_hbm.at[pl.ds(off, chunk)])
            return carry

        lax.fori_loop(0, steps, body, 0)

    return gather_kernel(table, idx)


def kernel(x, neighbors, max_iter, last_layer):
    del max_iter, last_layer
    n, d = x.shape
    m = neighbors.shape[0] // n
    x_norm = _normalize_call(x)
    table = jnp.concatenate(
        [x_norm, jnp.zeros((8, d), dtype=x_norm.dtype)], axis=0)
    nbr_t = _transpose_call(neighbors.reshape(n, m)).reshape(n * m)
    z = _gather_call(table, nbr_t)
    u = _routing_call(z.reshape(m, n, d), x_norm)
    return u
